# baseline (speedup 1.0000x reference)
import jax
import jax.numpy as jnp
from jax import lax
from jax.experimental import pallas as pl
from jax.experimental.pallas import tpu as pltpu
from jax.experimental.pallas import tpu_sc as plsc

NUM_WORKERS = 32
NUM_ROWS = 1000000
BATCH = 16384
B_PER_W = BATCH // NUM_WORKERS
EMBED = 64
LANES = 16
N_SEMS = 4

N_ROUNDS = 8
ROUND = B_PER_W // N_ROUNDS


def _body(uid_hbm, iid_hbm, utab_hbm, itab_hbm, out_hbm,
          idx_u, idx_i, u_buf, i_buf, out_v, *sems):
    wid = lax.axis_index("s") * 2 + lax.axis_index("c")
    iota = lax.iota(jnp.int32, LANES)

    pltpu.sync_copy(uid_hbm.at[pl.ds(wid * B_PER_W, B_PER_W)], idx_u)
    pltpu.sync_copy(iid_hbm.at[pl.ds(wid * B_PER_W, B_PER_W)], idx_i)

    def round_body(r, _):
        base = r * ROUND

        def fire_body(g, _):
            uvec = idx_u[pl.ds(base + g * LANES, LANES)]
            ivec = idx_i[pl.ds(base + g * LANES, LANES)]
            for l in range(LANES):
                j = g * LANES + l
                sem = sems[l % N_SEMS]
                pltpu.async_copy(utab_hbm.at[uvec[l]], u_buf.at[j], sem)
                pltpu.async_copy(itab_hbm.at[ivec[l]], i_buf.at[j], sem)
            return 0

        lax.fori_loop(0, ROUND // LANES, fire_body, 0)
        n = 2 * (ROUND // N_SEMS)
        for q in range(N_SEMS):
            pltpu.make_async_copy(
                utab_hbm.at[pl.ds(0, n)], u_buf.at[pl.ds(0, n)], sems[q]
            ).wait()

        def chunk_body(c, _):
            evec = c * LANES + iota
            acc = jnp.zeros((LANES,), jnp.float32)
            cvec = iota
            for _d in range(EMBED):
                u = plsc.load_gather(u_buf, [evec, cvec])
                v = plsc.load_gather(i_buf, [evec, cvec])
                acc = acc + u * v
                cvec = (cvec + 1) & (EMBED - 1)
            out_v[pl.ds(base + c * LANES, LANES)] = acc
            return 0

        lax.fori_loop(0, ROUND // LANES, chunk_body, 0)
        return 0

    lax.fori_loop(0, N_ROUNDS, round_body, 0)

    pltpu.sync_copy(out_v, out_hbm.at[pl.ds(wid * B_PER_W, B_PER_W)])


@jax.jit
def kernel(user_ids, item_ids, user_table, item_table):
    uids = user_ids.astype(jnp.int32)
    iids = item_ids.astype(jnp.int32)
    mesh = plsc.VectorSubcoreMesh(core_axis_name="c", subcore_axis_name="s")
    run = pl.kernel(
        _body,
        out_type=jax.ShapeDtypeStruct((BATCH,), jnp.float32),
        mesh=mesh,
        compiler_params=pltpu.CompilerParams(needs_layout_passes=False),
        scratch_types=[
            pltpu.VMEM((B_PER_W,), jnp.int32),
            pltpu.VMEM((B_PER_W,), jnp.int32),
            pltpu.VMEM((ROUND, EMBED), jnp.float32),
            pltpu.VMEM((ROUND, EMBED), jnp.float32),
            pltpu.VMEM((B_PER_W,), jnp.float32),
        ] + [pltpu.SemaphoreType.DMA] * N_SEMS,
    )
    return run(uids, iids, user_table, item_table)
